# baseline (speedup 1.0000x reference)
import functools
import jax
import jax.numpy as jnp
from jax import lax
from jax.experimental import pallas as pl
from jax.experimental.pallas import tpu as pltpu
from jax.experimental.pallas import tpu_sc as plsc

D = 128
DW = 144
NS = 16
NC = 2
LANES = 16


def _mm_kernel(x_ref, w_ref, o_ref):
    o_ref[...] = jnp.dot(x_ref[...], w_ref[...],
                         preferred_element_type=jnp.float32)


def _mm(x, w, bm=2048):
    M, K = x.shape
    _, N = w.shape
    return pl.pallas_call(
        _mm_kernel,
        grid=(pl.cdiv(M, bm),),
        in_specs=[
            pl.BlockSpec((bm, K), lambda i: (i, 0)),
            pl.BlockSpec((K, N), lambda i: (0, 0)),
        ],
        out_specs=pl.BlockSpec((bm, N), lambda i: (i, 0)),
        out_shape=jax.ShapeDtypeStruct((M, N), jnp.float32),
    )(x, w)


def _fin_kernel(acc_ref, o_ref):
    acc = acc_ref[...]
    num = acc[:, :D]
    den = acc[:, D:D + 1]
    x = num / (den + 1e-9)
    o_ref[...] = jnp.where(x > 0, x, jnp.exp(jnp.minimum(x, 0.0)) - 1.0)


def _finalize(acc, bm=2048):
    M = acc.shape[0]
    return pl.pallas_call(
        _fin_kernel,
        grid=(pl.cdiv(M, bm),),
        in_specs=[pl.BlockSpec((bm, DW), lambda i: (i, 0))],
        out_specs=pl.BlockSpec((bm, D), lambda i: (i, 0)),
        out_shape=jax.ShapeDtypeStruct((M, D), jnp.float32),
    )(acc)


def _make_sc_egat(Nsrc, E, Ndst, R, NP):
    assert NC * NP * R == Ndst
    assert E % NS == 0
    ES = E // NS
    SEG = 4000
    assert ES % SEG == 0 and SEG % LANES == 0
    NSEG = ES // SEG
    NZB = (R + 8) // 8
    NOB = R // 8
    assert R % 8 == 0

    mesh = plsc.VectorSubcoreMesh(core_axis_name="c", subcore_axis_name="s")

    @functools.partial(
        pl.kernel,
        mesh=mesh,
        out_type=jax.ShapeDtypeStruct((Ndst, DW), jnp.float32),
        scratch_types=[
            pltpu.VMEM((SEG,), jnp.int32),
            pltpu.VMEM((SEG,), jnp.int32),
            pltpu.VMEM((SEG + LANES,), jnp.int32),
            pltpu.VMEM((SEG + LANES,), jnp.int32),
            pltpu.VMEM((SEG + LANES,), jnp.int32),
            pltpu.VMEM((LANES, D), jnp.float32),
            pltpu.VMEM((LANES, D), jnp.float32),
            pltpu.VMEM((LANES, D), jnp.float32),
            pltpu.VMEM((LANES, DW), jnp.float32),
            pltpu.VMEM((D,), jnp.float32),
            pltpu.VMEM((8, DW), jnp.float32),
            pltpu.VMEM_SHARED((R + 8, DW), jnp.float32),
            pltpu.SemaphoreType.DMA,
            pltpu.SemaphoreType.DMA,
        ],
        compiler_params=pltpu.CompilerParams(use_tc_tiling_on_sc=False,
                                             needs_layout_passes=False),
    )
    def egat(z_hbm, ez_hbm, src_hbm, dst_hbm, a_hbm, acc_hbm,
             dseg, sseg, cid, csrc, cdst, zs, zd, ezb, msg, av, zrow,
             acc_sp, gsem, ssem):
        c = lax.axis_index("c")
        s = lax.axis_index("s")
        ebase = s * ES
        iota = lax.iota(jnp.int32, LANES)

        pltpu.sync_copy(a_hbm, av)

        zero16i = jnp.zeros((LANES,), jnp.int32)
        zero16f = jnp.zeros((LANES,), jnp.float32)

        def init_cbufs(i, _):
            cid[pl.ds(i * LANES, LANES)] = zero16i
            csrc[pl.ds(i * LANES, LANES)] = zero16i
            cdst[pl.ds(i * LANES, LANES)] = zero16i
            return 0
        lax.fori_loop(0, (SEG + LANES) // LANES, init_cbufs, 0)

        for r in range(8):
            for q in range(DW // LANES):
                plsc.store_scatter(zrow, [jnp.full((LANES,), r, jnp.int32),
                                          q * LANES + iota], zero16f)
        for r in range(LANES):
            for q in range(D // LANES, DW // LANES):
                plsc.store_scatter(msg, [jnp.full((LANES,), r, jnp.int32),
                                         q * LANES + iota], zero16f)

        def pass_body(p, _):
            base = c * (NP * R) + p * R

            def zero_blk(k, _):
                b = s + k * NS

                @pl.when(b < NZB)
                def _():
                    pltpu.sync_copy(zrow, acc_sp.at[pl.ds(b * 8, 8), :])
                return 0
            lax.fori_loop(0, pl.cdiv(NZB, NS), zero_blk, 0)
            plsc.subcore_barrier()

            def seg_body(g, _):
                pltpu.sync_copy(dst_hbm.at[pl.ds(ebase + g * SEG, SEG)], dseg)
                pltpu.sync_copy(src_hbm.at[pl.ds(ebase + g * SEG, SEG)], sseg)

                def scan_body(i, n):
                    dvec = dseg[pl.ds(i * LANES, LANES)]
                    svec = sseg[pl.ds(i * LANES, LANES)]
                    m = (dvec >= base) & (dvec < base + R)
                    ids = g * SEG + i * LANES + iota
                    plsc.store_compressed(cid.at[pl.ds(n, LANES)], ids,
                                          mask=m)
                    plsc.store_compressed(csrc.at[pl.ds(n, LANES)], svec,
                                          mask=m)
                    plsc.store_compressed(cdst.at[pl.ds(n, LANES)], dvec,
                                          mask=m)
                    return n + jnp.sum(m.astype(jnp.int32))
                n_match = lax.fori_loop(0, SEG // LANES, scan_body, 0)
                nc = (n_match + LANES - 1) // LANES

                def chunk_body(ci, _):
                    off = ci * LANES
                    v = cid[pl.ds(off, LANES)]
                    srcv = csrc[pl.ds(off, LANES)]
                    dstv = cdst[pl.ds(off, LANES)]
                    geid = v + ebase
                    g1 = pltpu.async_copy(z_hbm.at[srcv], zs, gsem)
                    g2 = pltpu.async_copy(z_hbm.at[dstv], zd, gsem)
                    g3 = pltpu.async_copy(ez_hbm.at[geid], ezb, gsem)
                    g1.wait()
                    g2.wait()
                    g3.wait()

                    acc = jnp.zeros((LANES,), jnp.float32)
                    for j in range(D):
                        cj = jnp.full((LANES,), j, jnp.int32)
                        f = (plsc.load_gather(zs, [iota, cj])
                             + plsc.load_gather(zd, [iota, cj])
                             + plsc.load_gather(ezb, [iota, cj]))
                        lr = jnp.where(f > 0, f, f * 0.2)
                        aq = av[pl.ds((j // LANES) * LANES, LANES)]
                        aj = jnp.take_along_axis(
                            aq, jnp.full((LANES,), j % LANES, jnp.int32),
                            axis=0)
                        acc = acc + lr * aj
                    ex = jnp.exp(acc)

                    valid = (off + iota) < n_match
                    sidx = jnp.where(valid, dstv - base, R)

                    for j in range(D):
                        cj = jnp.full((LANES,), j, jnp.int32)
                        mz = plsc.load_gather(zs, [iota, cj]) * ex
                        plsc.store_scatter(msg, [iota, cj], mz)
                    plsc.store_scatter(msg, [iota, jnp.full((LANES,), D,
                                                            jnp.int32)], ex)

                    sc = pltpu.async_copy(msg, acc_sp.at[sidx], ssem,
                                          add=True)
                    sc.wait()
                    return 0
                lax.fori_loop(0, nc, chunk_body, 0)
                return 0
            lax.fori_loop(0, NSEG, seg_body, 0)
            plsc.subcore_barrier()

            def out_blk(k, _):
                b = s + k * NS

                @pl.when(b < NOB)
                def _():
                    pltpu.sync_copy(
                        acc_sp.at[pl.ds(b * 8, 8), :],
                        acc_hbm.at[pl.ds(base + b * 8, 8), :])
                return 0
            lax.fori_loop(0, pl.cdiv(NOB, NS), out_blk, 0)
            plsc.subcore_barrier()
            return 0

        lax.fori_loop(0, NP, pass_body, 0)

    return egat


_sc_egat_l1 = _make_sc_egat(10000, 320000, 10000, 5000, 1)
_sc_egat_l2 = _make_sc_egat(320000, 320000, 320000, 10000, 16)


def kernel(h, edge_features, edge_ft_upper, edge_index, edge_index_upper,
           Wn1, We1, a1, Wn2, We2, a2):
    src1, dst1 = edge_index[0], edge_index[1]
    src2, dst2 = edge_index_upper[0], edge_index_upper[1]

    z1 = _mm(h, Wn1)
    ez1 = _mm(edge_features, We1)
    acc1 = _sc_egat_l1(z1, ez1, src1, dst1, a1.reshape(-1))
    node_embeddings = _finalize(acc1)

    z2 = _mm(edge_features, Wn2)
    ez2 = _mm(edge_ft_upper, We2)
    acc2 = _sc_egat_l2(z2, ez2, src2, dst2, a2.reshape(-1))
    edge_embeddings = _finalize(acc2)

    return (node_embeddings, edge_embeddings)


# speedup vs baseline: 1.0941x; 1.0941x over previous
import functools
import jax
import jax.numpy as jnp
from jax import lax
from jax.experimental import pallas as pl
from jax.experimental.pallas import tpu as pltpu
from jax.experimental.pallas import tpu_sc as plsc

D = 128
DW = 144
NS = 16
NC = 2
LANES = 16


def _mm_kernel(x_ref, w_ref, o_ref):
    o_ref[...] = jnp.dot(x_ref[...], w_ref[...],
                         preferred_element_type=jnp.float32)


def _mm(x, w, bm=2048):
    M, K = x.shape
    _, N = w.shape
    return pl.pallas_call(
        _mm_kernel,
        grid=(pl.cdiv(M, bm),),
        in_specs=[
            pl.BlockSpec((bm, K), lambda i: (i, 0)),
            pl.BlockSpec((K, N), lambda i: (0, 0)),
        ],
        out_specs=pl.BlockSpec((bm, N), lambda i: (i, 0)),
        out_shape=jax.ShapeDtypeStruct((M, N), jnp.float32),
    )(x, w)


def _fin_kernel(acc_ref, o_ref):
    acc = acc_ref[...]
    num = acc[:, :D]
    den = acc[:, D:D + 1]
    x = num / (den + 1e-9)
    o_ref[...] = jnp.where(x > 0, x, jnp.exp(jnp.minimum(x, 0.0)) - 1.0)


def _finalize(acc, bm=2048):
    M = acc.shape[0]
    return pl.pallas_call(
        _fin_kernel,
        grid=(pl.cdiv(M, bm),),
        in_specs=[pl.BlockSpec((bm, DW), lambda i: (i, 0))],
        out_specs=pl.BlockSpec((bm, D), lambda i: (i, 0)),
        out_shape=jax.ShapeDtypeStruct((M, D), jnp.float32),
    )(acc)


def _make_sc_egat(Nsrc, E, Ndst, R, NP):
    assert NC * NP * R == Ndst
    assert E % NS == 0
    ES = E // NS
    SEG = 4000
    assert ES % SEG == 0 and SEG % LANES == 0
    NSEG = ES // SEG
    OB = 40
    NZB = R // OB + 1
    NOB = R // OB
    assert R % OB == 0

    mesh = plsc.VectorSubcoreMesh(core_axis_name="c", subcore_axis_name="s")

    @functools.partial(
        pl.kernel,
        mesh=mesh,
        out_type=jax.ShapeDtypeStruct((Ndst, DW), jnp.float32),
        scratch_types=[
            pltpu.VMEM((SEG,), jnp.int32),
            pltpu.VMEM((SEG,), jnp.int32),
            pltpu.VMEM((SEG + LANES,), jnp.int32),
            pltpu.VMEM((SEG + LANES,), jnp.int32),
            pltpu.VMEM((SEG + LANES,), jnp.int32),
            pltpu.VMEM((LANES, D), jnp.float32),
            pltpu.VMEM((LANES, D), jnp.float32),
            pltpu.VMEM((LANES, D), jnp.float32),
            pltpu.VMEM((LANES, DW), jnp.float32),
            pltpu.VMEM((D,), jnp.float32),
            pltpu.VMEM((40, DW), jnp.float32),
            pltpu.VMEM_SHARED((R + 40, DW), jnp.float32),
            pltpu.SemaphoreType.DMA,
            pltpu.SemaphoreType.DMA,
        ],
        compiler_params=pltpu.CompilerParams(use_tc_tiling_on_sc=False,
                                             needs_layout_passes=False),
    )
    def egat(z_hbm, ez_hbm, src_hbm, dst_hbm, a_hbm, acc_hbm,
             dseg, sseg, cid, csrc, cdst, zs, zd, ezb, msg, av, zrow,
             acc_sp, gsem, ssem):
        c = lax.axis_index("c")
        s = lax.axis_index("s")
        ebase = s * ES
        iota = lax.iota(jnp.int32, LANES)

        pltpu.sync_copy(a_hbm, av)

        zero16i = jnp.zeros((LANES,), jnp.int32)
        zero16f = jnp.zeros((LANES,), jnp.float32)

        def init_cbufs(i, _):
            cid[pl.ds(i * LANES, LANES)] = zero16i
            csrc[pl.ds(i * LANES, LANES)] = zero16i
            cdst[pl.ds(i * LANES, LANES)] = zero16i
            return 0
        lax.fori_loop(0, (SEG + LANES) // LANES, init_cbufs, 0)

        def zr_init(r, _):
            rv = jnp.zeros((LANES,), jnp.int32) + r
            for q in range(DW // LANES):
                plsc.store_scatter(zrow, [rv, q * LANES + iota], zero16f)
            return 0
        lax.fori_loop(0, OB, zr_init, 0)
        for r in range(LANES):
            for q in range(D // LANES, DW // LANES):
                plsc.store_scatter(msg, [jnp.full((LANES,), r, jnp.int32),
                                         q * LANES + iota], zero16f)

        def pass_body(p, _):
            base = c * (NP * R) + p * R

            def zero_blk(k, _):
                b = s + k * NS

                @pl.when(b < NZB)
                def _():
                    pltpu.sync_copy(zrow, acc_sp.at[pl.ds(b * OB, OB), :])
                return 0
            lax.fori_loop(0, pl.cdiv(NZB, NS), zero_blk, 0)
            plsc.subcore_barrier()

            def seg_body(g, _):
                pltpu.sync_copy(dst_hbm.at[pl.ds(ebase + g * SEG, SEG)], dseg)
                pltpu.sync_copy(src_hbm.at[pl.ds(ebase + g * SEG, SEG)], sseg)

                def scan_body(i, n):
                    dvec = dseg[pl.ds(i * LANES, LANES)]
                    svec = sseg[pl.ds(i * LANES, LANES)]
                    m = (dvec >= base) & (dvec < base + R)
                    ids = g * SEG + i * LANES + iota
                    plsc.store_compressed(cid.at[pl.ds(n, LANES)], ids,
                                          mask=m)
                    plsc.store_compressed(csrc.at[pl.ds(n, LANES)], svec,
                                          mask=m)
                    plsc.store_compressed(cdst.at[pl.ds(n, LANES)], dvec,
                                          mask=m)
                    return n + jnp.sum(m.astype(jnp.int32))
                n_match = lax.fori_loop(0, SEG // LANES, scan_body, 0)
                nc = (n_match + LANES - 1) // LANES

                def chunk_body(ci, _):
                    off = ci * LANES
                    v = cid[pl.ds(off, LANES)]
                    srcv = csrc[pl.ds(off, LANES)]
                    dstv = cdst[pl.ds(off, LANES)]
                    geid = v + ebase
                    g1 = pltpu.async_copy(z_hbm.at[srcv], zs, gsem)
                    g2 = pltpu.async_copy(z_hbm.at[dstv], zd, gsem)
                    g3 = pltpu.async_copy(ez_hbm.at[geid], ezb, gsem)
                    g1.wait()
                    g2.wait()
                    g3.wait()

                    acc = jnp.zeros((LANES,), jnp.float32)
                    for j in range(D):
                        cj = jnp.full((LANES,), j, jnp.int32)
                        f = (plsc.load_gather(zs, [iota, cj])
                             + plsc.load_gather(zd, [iota, cj])
                             + plsc.load_gather(ezb, [iota, cj]))
                        lr = jnp.where(f > 0, f, f * 0.2)
                        aq = av[pl.ds((j // LANES) * LANES, LANES)]
                        aj = jnp.take_along_axis(
                            aq, jnp.full((LANES,), j % LANES, jnp.int32),
                            axis=0)
                        acc = acc + lr * aj
                    ex = jnp.exp(acc)

                    valid = (off + iota) < n_match
                    sidx = jnp.where(valid, dstv - base, R)

                    for j in range(D):
                        cj = jnp.full((LANES,), j, jnp.int32)
                        mz = plsc.load_gather(zs, [iota, cj]) * ex
                        plsc.store_scatter(msg, [iota, cj], mz)
                    plsc.store_scatter(msg, [iota, jnp.full((LANES,), D,
                                                            jnp.int32)], ex)

                    sc = pltpu.async_copy(msg, acc_sp.at[sidx], ssem,
                                          add=True)
                    sc.wait()
                    return 0
                lax.fori_loop(0, nc, chunk_body, 0)
                return 0
            lax.fori_loop(0, NSEG, seg_body, 0)
            plsc.subcore_barrier()

            def out_blk(k, _):
                b = s + k * NS

                @pl.when(b < NOB)
                def _():
                    pltpu.sync_copy(
                        acc_sp.at[pl.ds(b * OB, OB), :],
                        acc_hbm.at[pl.ds(base + b * OB, OB), :])
                return 0
            lax.fori_loop(0, pl.cdiv(NOB, NS), out_blk, 0)
            plsc.subcore_barrier()
            return 0

        lax.fori_loop(0, NP, pass_body, 0)

    return egat


_sc_egat_l1 = _make_sc_egat(10000, 320000, 10000, 5000, 1)
_sc_egat_l2 = _make_sc_egat(320000, 320000, 320000, 10000, 16)


def kernel(h, edge_features, edge_ft_upper, edge_index, edge_index_upper,
           Wn1, We1, a1, Wn2, We2, a2):
    src1, dst1 = edge_index[0], edge_index[1]
    src2, dst2 = edge_index_upper[0], edge_index_upper[1]

    z1 = _mm(h, Wn1)
    ez1 = _mm(edge_features, We1)
    acc1 = _sc_egat_l1(z1, ez1, src1, dst1, a1.reshape(-1))
    node_embeddings = _finalize(acc1)

    z2 = _mm(edge_features, Wn2)
    ez2 = _mm(edge_ft_upper, We2)
    acc2 = _sc_egat_l2(z2, ez2, src2, dst2, a2.reshape(-1))
    edge_embeddings = _finalize(acc2)

    return (node_embeddings, edge_embeddings)


# speedup vs baseline: 2.7020x; 2.4695x over previous
import functools
import jax
import jax.numpy as jnp
from jax import lax
from jax.experimental import pallas as pl
from jax.experimental.pallas import tpu as pltpu
from jax.experimental.pallas import tpu_sc as plsc

D = 128
DW = 144
NS = 16
NC = 2
LANES = 16


def _mm_kernel(x_ref, w_ref, o_ref):
    o_ref[...] = jnp.dot(x_ref[...], w_ref[...],
                         preferred_element_type=jnp.float32)


def _mm(x, w, bm=2048):
    M, K = x.shape
    _, N = w.shape
    return pl.pallas_call(
        _mm_kernel,
        grid=(pl.cdiv(M, bm),),
        in_specs=[
            pl.BlockSpec((bm, K), lambda i: (i, 0)),
            pl.BlockSpec((K, N), lambda i: (0, 0)),
        ],
        out_specs=pl.BlockSpec((bm, N), lambda i: (i, 0)),
        out_shape=jax.ShapeDtypeStruct((M, N), jnp.float32),
    )(x, w)


def _fin_kernel(acc_ref, o_ref):
    acc = acc_ref[...]
    num = acc[:, :D]
    den = acc[:, D:D + 1]
    x = num / (den + 1e-9)
    o_ref[...] = jnp.where(x > 0, x, jnp.exp(jnp.minimum(x, 0.0)) - 1.0)


def _finalize(acc, bm=2048):
    M = acc.shape[0]
    return pl.pallas_call(
        _fin_kernel,
        grid=(pl.cdiv(M, bm),),
        in_specs=[pl.BlockSpec((bm, DW), lambda i: (i, 0))],
        out_specs=pl.BlockSpec((bm, D), lambda i: (i, 0)),
        out_shape=jax.ShapeDtypeStruct((M, D), jnp.float32),
    )(acc)


def _make_sc_egat(Nsrc, E, Ndst, R, NP):
    assert NC * NP * R == Ndst
    assert E % NS == 0
    ES = E // NS
    SEG = 4000
    assert ES % SEG == 0 and SEG % LANES == 0
    NSEG = ES // SEG
    OB = 40
    NZB = R // OB + 1
    NOB = R // OB
    assert R % OB == 0

    mesh = plsc.VectorSubcoreMesh(core_axis_name="c", subcore_axis_name="s")

    @functools.partial(
        pl.kernel,
        mesh=mesh,
        out_type=jax.ShapeDtypeStruct((Ndst, DW), jnp.float32),
        scratch_types=[
            pltpu.VMEM((SEG,), jnp.int32),
            pltpu.VMEM((SEG,), jnp.int32),
            pltpu.VMEM((SEG + LANES,), jnp.int32),
            pltpu.VMEM((SEG + LANES,), jnp.int32),
            pltpu.VMEM((SEG + LANES,), jnp.int32),
            pltpu.VMEM((LANES, D), jnp.float32),
            pltpu.VMEM((LANES, D), jnp.float32),
            pltpu.VMEM((LANES, D), jnp.float32),
            pltpu.VMEM((LANES, DW), jnp.float32),
            pltpu.VMEM((LANES, LANES), jnp.float32),
            pltpu.VMEM((D,), jnp.float32),
            pltpu.VMEM((40, DW), jnp.float32),
            pltpu.VMEM_SHARED((R + 40, DW), jnp.float32),
            pltpu.SemaphoreType.DMA,
            pltpu.SemaphoreType.DMA,
        ],
        compiler_params=pltpu.CompilerParams(use_tc_tiling_on_sc=False,
                                             needs_layout_passes=False),
    )
    def egat(z_hbm, ez_hbm, src_hbm, dst_hbm, a_hbm, acc_hbm,
             dseg, sseg, cid, csrc, cdst, zs, zd, ezb, msg, tmp, av, zrow,
             acc_sp, gsem, ssem):
        c = lax.axis_index("c")
        s = lax.axis_index("s")
        ebase = s * ES
        iota = lax.iota(jnp.int32, LANES)

        pltpu.sync_copy(a_hbm, av)

        zero16i = jnp.zeros((LANES,), jnp.int32)
        zero16f = jnp.zeros((LANES,), jnp.float32)

        def init_cbufs(i, _):
            cid[pl.ds(i * LANES, LANES)] = zero16i
            csrc[pl.ds(i * LANES, LANES)] = zero16i
            cdst[pl.ds(i * LANES, LANES)] = zero16i
            return 0
        lax.fori_loop(0, (SEG + LANES) // LANES, init_cbufs, 0)

        def zr_init(r, _):
            rv = jnp.zeros((LANES,), jnp.int32) + r
            for q in range(DW // LANES):
                plsc.store_scatter(zrow, [rv, q * LANES + iota], zero16f)
            return 0
        lax.fori_loop(0, OB, zr_init, 0)
        for r in range(LANES):
            for q in range(D // LANES, DW // LANES):
                plsc.store_scatter(msg, [jnp.full((LANES,), r, jnp.int32),
                                         q * LANES + iota], zero16f)

        def pass_body(p, _):
            base = c * (NP * R) + p * R

            def zero_blk(k, _):
                b = s + k * NS

                @pl.when(b < NZB)
                def _():
                    pltpu.sync_copy(zrow, acc_sp.at[pl.ds(b * OB, OB), :])
                return 0
            lax.fori_loop(0, pl.cdiv(NZB, NS), zero_blk, 0)
            plsc.subcore_barrier()

            def seg_body(g, _):
                pltpu.sync_copy(dst_hbm.at[pl.ds(ebase + g * SEG, SEG)], dseg)
                pltpu.sync_copy(src_hbm.at[pl.ds(ebase + g * SEG, SEG)], sseg)

                def scan_body(i, n):
                    dvec = dseg[pl.ds(i * LANES, LANES)]
                    svec = sseg[pl.ds(i * LANES, LANES)]
                    m = (dvec >= base) & (dvec < base + R)
                    ids = g * SEG + i * LANES + iota
                    plsc.store_compressed(cid.at[pl.ds(n, LANES)], ids,
                                          mask=m)
                    plsc.store_compressed(csrc.at[pl.ds(n, LANES)], svec,
                                          mask=m)
                    plsc.store_compressed(cdst.at[pl.ds(n, LANES)], dvec,
                                          mask=m)
                    return n + jnp.sum(m.astype(jnp.int32))
                n_match = lax.fori_loop(0, SEG // LANES, scan_body, 0)
                nc = (n_match + LANES - 1) // LANES

                def chunk_body(ci, _):
                    off = ci * LANES
                    v = cid[pl.ds(off, LANES)]
                    srcv = csrc[pl.ds(off, LANES)]
                    dstv = cdst[pl.ds(off, LANES)]
                    geid = v + ebase
                    g1 = pltpu.async_copy(z_hbm.at[srcv], zs, gsem)
                    g2 = pltpu.async_copy(z_hbm.at[dstv], zd, gsem)
                    g3 = pltpu.async_copy(ez_hbm.at[geid], ezb, gsem)
                    g1.wait()
                    g2.wait()
                    g3.wait()

                    for i in range(LANES):
                        pacc = jnp.zeros((LANES,), jnp.float32)
                        for q in range(D // LANES):
                            sl = pl.ds(q * LANES, LANES)
                            f = zs[i, sl] + zd[i, sl] + ezb[i, sl]
                            lr = jnp.where(f > 0, f, f * 0.2)
                            pacc = pacc + lr * av[sl]
                        tmp[i, :] = pacc
                    acc = jnp.zeros((LANES,), jnp.float32)
                    for q in range(LANES):
                        acc = acc + plsc.load_gather(
                            tmp, [iota, jnp.full((LANES,), q, jnp.int32)])
                    ex = jnp.exp(acc)

                    valid = (off + iota) < n_match
                    sidx = jnp.where(valid, dstv - base, R)

                    for i in range(LANES):
                        exi = jnp.take_along_axis(
                            ex, jnp.full((LANES,), i, jnp.int32), axis=0)
                        for q in range(D // LANES):
                            sl = pl.ds(q * LANES, LANES)
                            msg[i, sl] = zs[i, sl] * exi
                    plsc.store_scatter(msg, [iota, jnp.full((LANES,), D,
                                                            jnp.int32)], ex)

                    sc = pltpu.async_copy(msg, acc_sp.at[sidx], ssem,
                                          add=True)
                    sc.wait()
                    return 0
                lax.fori_loop(0, nc, chunk_body, 0)
                return 0
            lax.fori_loop(0, NSEG, seg_body, 0)
            plsc.subcore_barrier()

            def out_blk(k, _):
                b = s + k * NS

                @pl.when(b < NOB)
                def _():
                    pltpu.sync_copy(
                        acc_sp.at[pl.ds(b * OB, OB), :],
                        acc_hbm.at[pl.ds(base + b * OB, OB), :])
                return 0
            lax.fori_loop(0, pl.cdiv(NOB, NS), out_blk, 0)
            plsc.subcore_barrier()
            return 0

        lax.fori_loop(0, NP, pass_body, 0)

    return egat


_sc_egat_l1 = _make_sc_egat(10000, 320000, 10000, 5000, 1)
_sc_egat_l2 = _make_sc_egat(320000, 320000, 320000, 10000, 16)


def kernel(h, edge_features, edge_ft_upper, edge_index, edge_index_upper,
           Wn1, We1, a1, Wn2, We2, a2):
    src1, dst1 = edge_index[0], edge_index[1]
    src2, dst2 = edge_index_upper[0], edge_index_upper[1]

    z1 = _mm(h, Wn1)
    ez1 = _mm(edge_features, We1)
    acc1 = _sc_egat_l1(z1, ez1, src1, dst1, a1.reshape(-1))
    node_embeddings = _finalize(acc1)

    z2 = _mm(edge_features, Wn2)
    ez2 = _mm(edge_ft_upper, We2)
    acc2 = _sc_egat_l2(z2, ez2, src2, dst2, a2.reshape(-1))
    edge_embeddings = _finalize(acc2)

    return (node_embeddings, edge_embeddings)
